# baseline (speedup 1.0000x reference)
import functools
import math

import jax
import jax.numpy as jnp
from jax import lax
from jax.experimental import pallas as pl
from jax.experimental.pallas import tpu as pltpu
from jax.experimental.pallas import tpu_sc as plsc

B, S, D, V, T, DFF = 8, 512, 1024, 32000, 64, 2048
N = B * S

_HI = jax.lax.Precision.HIGHEST
_NEG = -1e30


def _mm(a, b, precision=jax.lax.Precision.DEFAULT):
    return lax.dot_general(a, b, (((a.ndim - 1,), (0,)), ((), ())),
                           precision=precision,
                           preferred_element_type=jnp.float32)


def _mm1(a, b, dims=None):
    if dims is None:
        dims = (((a.ndim - 1,), (0,)), ((), ()))
    return lax.dot_general(a.astype(jnp.bfloat16), b.astype(jnp.bfloat16),
                           dims, preferred_element_type=jnp.float32)


def _emb_gather(table, idx):
    info = plsc.get_sparse_core_info()
    nw = info.num_cores * info.num_subcores
    b_per_w = N // nw
    ch = 64
    mesh = plsc.VectorSubcoreMesh(core_axis_name="c", subcore_axis_name="s")

    @functools.partial(
        pl.kernel, mesh=mesh,
        out_type=jax.ShapeDtypeStruct((N, D), jnp.float32),
        scratch_types=[
            pltpu.VMEM((b_per_w,), jnp.int32),
            pltpu.VMEM((ch, D), jnp.float32),
            pltpu.SemaphoreType.DMA,
        ],
    )
    def k(table_hbm, idx_hbm, out_hbm, idx_v, rows_v, sem):
        wid = lax.axis_index("s") * info.num_cores + lax.axis_index("c")
        base = wid * b_per_w
        pltpu.sync_copy(idx_hbm.at[pl.ds(base, b_per_w)], idx_v)
        for c in range(b_per_w // ch):
            pltpu.async_copy(table_hbm.at[idx_v.at[pl.ds(c * ch, ch)]],
                             rows_v, sem).wait()
            pltpu.sync_copy(rows_v, out_hbm.at[pl.ds(base + c * ch, ch)])

    return k(table, idx)


def _qkv(x, wq, wk, wv):
    rb = 512

    def body(x_ref, wq_ref, wk_ref, wv_ref, q_ref, k_ref, v_ref):
        xv = x_ref[...].astype(jnp.bfloat16)
        q_ref[...] = _mm1(xv, wq_ref[...]).astype(jnp.bfloat16)
        k_ref[...] = _mm1(xv, wk_ref[...]).astype(jnp.bfloat16)
        v_ref[...] = _mm1(xv, wv_ref[...])

    w_spec = pl.BlockSpec((D, D), lambda i: (0, 0))
    r_spec = pl.BlockSpec((rb, D), lambda i: (i, 0))
    return pl.pallas_call(
        body,
        grid=(N // rb,),
        in_specs=[r_spec, w_spec, w_spec, w_spec],
        out_specs=[r_spec, r_spec, r_spec],
        out_shape=[jax.ShapeDtypeStruct((N, D), jnp.bfloat16),
                   jax.ShapeDtypeStruct((N, D), jnp.bfloat16),
                   jax.ShapeDtypeStruct((N, D), jnp.float32)],
    )(x, wq, wk, wv)


def _attention(q, k, v, kvalid):
    bq, bk = 1024, 1024
    nq, nk = N // bq, N // bk
    scale = 1.0 / math.sqrt(D)

    def body(q_ref, k_ref, v_ref, kv_ref, o_ref, m_s, l_s, acc_s):
        j = pl.program_id(1)

        @pl.when(j == 0)
        def _():
            m_s[...] = jnp.full_like(m_s, -jnp.inf)
            l_s[...] = jnp.zeros_like(l_s)
            acc_s[...] = jnp.zeros_like(acc_s)

        s = lax.dot_general(q_ref[...], k_ref[...],
                            (((1,), (1,)), ((), ())),
                            preferred_element_type=jnp.float32)
        s = jnp.where(kv_ref[...] > 0.0, s * scale, -jnp.inf)
        m_old = m_s[...]
        l_old = l_s[...]
        m_new = jnp.maximum(m_old, jnp.max(s, axis=1, keepdims=True))
        corr = jnp.exp(jnp.where(m_old == m_new, 0.0, m_old - m_new))
        p = jnp.exp(s - m_new)
        l_new = corr * l_old + jnp.sum(p, axis=1, keepdims=True)
        tot = _mm1(p, v_ref[...]) + (corr * l_old) * acc_s[...]
        acc_s[...] = tot * (1.0 / l_new)
        m_s[...] = m_new
        l_s[...] = l_new

        @pl.when(j == nk - 1)
        def _():
            o_ref[...] = acc_s[...]

    return pl.pallas_call(
        body,
        grid=(nq, nk),
        in_specs=[
            pl.BlockSpec((bq, D), lambda i, j: (i, 0)),
            pl.BlockSpec((bk, D), lambda i, j: (j, 0)),
            pl.BlockSpec((bk, D), lambda i, j: (j, 0)),
            pl.BlockSpec((1, bk), lambda i, j: (0, j)),
        ],
        out_specs=pl.BlockSpec((bq, D), lambda i, j: (i, 0)),
        out_shape=jax.ShapeDtypeStruct((N, D), jnp.float32),
        scratch_shapes=[
            pltpu.VMEM((bq, 1), jnp.float32),
            pltpu.VMEM((bq, 1), jnp.float32),
            pltpu.VMEM((bq, D), jnp.float32),
        ],
        compiler_params=pltpu.CompilerParams(
            dimension_semantics=("parallel", "arbitrary")),
    )(q, k, v, kvalid)


def _layernorm(h):
    m = jnp.mean(h, axis=-1, keepdims=True)
    d = h - m
    var = jnp.mean(d * d, axis=-1, keepdims=True)
    return d / jnp.sqrt(var + 1e-5)


def _post(attn, x, wo, w1, w2, wemit, validf):
    rb = 512

    def body(a_ref, x_ref, wo_ref, w1_ref, w2_ref, we_ref, vf_ref, em_ref):
        h = _layernorm(x_ref[...] + _mm1(a_ref[...], wo_ref[...]))
        g = _mm1(h, w1_ref[...]).astype(jnp.bfloat16)
        r = jnp.maximum(g, jnp.bfloat16(0))
        f = _layernorm(h + _mm1(r, w2_ref[...]))
        f = f * vf_ref[...]
        em_ref[...] = _mm1(f, we_ref[...])

    return pl.pallas_call(
        body,
        grid=(N // rb,),
        in_specs=[
            pl.BlockSpec((rb, D), lambda i: (i, 0)),
            pl.BlockSpec((rb, D), lambda i: (i, 0)),
            pl.BlockSpec((D, D), lambda i: (0, 0)),
            pl.BlockSpec((D, DFF), lambda i: (0, 0)),
            pl.BlockSpec((DFF, D), lambda i: (0, 0)),
            pl.BlockSpec((D, T), lambda i: (0, 0)),
            pl.BlockSpec((rb, 1), lambda i: (i, 0)),
        ],
        out_specs=pl.BlockSpec((rb, T), lambda i: (i, 0)),
        out_shape=jax.ShapeDtypeStruct((N, T), jnp.float32),
    )(attn, x, wo, w1, w2, wemit, validf)


def _crf(em_t, y_t, vf_t, trans, start_r, end_r):
    def body(em_ref, y_ref, vf_ref, tr_ref, st_ref, en_ref,
             loss_ref, path_ref, bp_ref):
        trans_v = tr_ref[...]
        etrans = jnp.exp(trans_v)
        start = st_ref[...]
        end = en_ref[...]
        iota_bt = lax.broadcasted_iota(jnp.int32, (B, T), 1)

        em0 = em_ref[pl.ds(0, 1)][0]
        alpha0 = start + em0
        delta0 = alpha0

        def fwd(t, carry):
            alpha, delta = carry
            em_step = em_ref[pl.ds(t, 1)][0]
            vt = vf_ref[pl.ds(t, 1)].reshape(B, 1) > 0.0
            m = jnp.max(alpha, axis=1, keepdims=True)
            p = jnp.exp(alpha - m)
            acc = _mm(p, etrans, precision=_HI)
            na = jnp.log(acc) + m + em_step
            alpha = jnp.where(vt, na, alpha)
            s3 = delta[:, :, None] + trans_v[None, :, :]
            mx = jnp.max(s3, axis=1)
            ii = lax.broadcasted_iota(jnp.int32, (B, T, T), 1)
            hit = s3 >= mx[:, None, :]
            bp = jnp.min(jnp.where(hit, ii, T), axis=1)
            bp_ref[pl.ds(t, 1)] = jnp.where(vt, bp, iota_bt)[None]
            delta = jnp.where(vt, mx + em_step, delta)
            return alpha, delta

        alpha, delta = lax.fori_loop(1, S, fwd, (alpha0, delta0), unroll=2)

        em_all = em_ref[...]
        y = y_ref[...]
        vf = vf_ref[...]
        iota3 = lax.broadcasted_iota(jnp.int32, (S, B, T), 2)
        yoh = (iota3 == y[:, :, None]).astype(jnp.float32)
        em_y = jnp.sum(em_all * yoh, axis=2)
        g_em = jnp.sum(em_y * vf, axis=0)
        st_term = jnp.sum(start * (iota_bt == y[0][:, None]), axis=1)
        rows = _mm(yoh[:-1].reshape((S - 1) * B, T), trans_v, precision=_HI)
        sel = jnp.sum(rows * yoh[1:].reshape((S - 1) * B, T),
                      axis=1).reshape(S - 1, B)
        g_tr = jnp.sum(sel * vf[1:], axis=0)
        lengths = jnp.sum(vf, axis=0).astype(jnp.int32)
        iota_sb = lax.broadcasted_iota(jnp.int32, (S, B), 0)
        ohlen = (iota_sb == (lengths - 1)[None, :]).astype(jnp.int32)
        ylast = jnp.sum(y * ohlen, axis=0)
        en_term = jnp.sum(end * (iota_bt == ylast[:, None]), axis=1)
        gold = g_em + st_term + g_tr + en_term
        ae = alpha + end
        mz = jnp.max(ae, axis=1, keepdims=True)
        logz = mz[:, 0] + jnp.log(jnp.sum(jnp.exp(ae - mz), axis=1))
        loss_ref[...] = jnp.mean(logz - gold).reshape(1, 1)

        de = delta + end
        mx0 = jnp.max(de, axis=1, keepdims=True)
        last0 = jnp.min(jnp.where(de >= mx0, iota_bt, T),
                        axis=1, keepdims=True)
        path_ref[pl.ds(S - 1, 1)] = last0.reshape(1, B)

        def back(i, last):
            t = S - 2 - i
            bpt = bp_ref[pl.ds(t + 1, 1)][0]
            eq = (iota_bt == last).astype(jnp.int32)
            nxt = jnp.sum(bpt * eq, axis=1, keepdims=True)
            path_ref[pl.ds(t, 1)] = nxt.reshape(1, B)
            return nxt

        lax.fori_loop(0, S - 1, back, last0, unroll=4)

    loss, path_t = pl.pallas_call(
        body,
        in_specs=[pl.BlockSpec(memory_space=pltpu.VMEM)] * 6,
        out_specs=[pl.BlockSpec(memory_space=pltpu.VMEM)] * 2,
        out_shape=[jax.ShapeDtypeStruct((1, 1), jnp.float32),
                   jax.ShapeDtypeStruct((S, B), jnp.int32)],
        scratch_shapes=[pltpu.VMEM((S, B, T), jnp.int32)],
    )(em_t, y_t, vf_t, trans, start_r, end_r)
    return loss, path_t


def kernel(src, y, mask, use_gpu, embedding_matrix, Wq, Wk, Wv, Wo,
           W1, W2, W_emit, trans, start, end):
    valid = jnp.logical_not(mask)
    valid_flat = valid.reshape(-1)
    validf = valid_flat.astype(jnp.float32)
    src_flat = src.reshape(-1).astype(jnp.int32)

    x = _emb_gather(embedding_matrix, src_flat)
    q, k, v = _qkv(x, Wq, Wk, Wv)
    attn = _attention(q, k, v, validf.reshape(1, N))
    em = _post(attn, x, Wo, W1, W2, W_emit, validf.reshape(N, 1))

    em_t = em.reshape(B, S, T).transpose(1, 0, 2)
    y_t = y.astype(jnp.int32).transpose(1, 0)
    vf_t = validf.reshape(B, S).transpose(1, 0)
    loss, path_t = _crf(em_t, y_t, vf_t, trans,
                        start.reshape(1, T), end.reshape(1, T))
    return loss[0, 0], path_t.transpose(1, 0)


# speedup vs baseline: 1.1017x; 1.1017x over previous
import functools
import math

import jax
import jax.numpy as jnp
from jax import lax
from jax.experimental import pallas as pl
from jax.experimental.pallas import tpu as pltpu
from jax.experimental.pallas import tpu_sc as plsc

B, S, D, V, T, DFF = 8, 512, 1024, 32000, 64, 2048
N = B * S

_HI = jax.lax.Precision.HIGHEST
_NEG = -1e30


def _mm(a, b, precision=jax.lax.Precision.DEFAULT):
    return lax.dot_general(a, b, (((a.ndim - 1,), (0,)), ((), ())),
                           precision=precision,
                           preferred_element_type=jnp.float32)


def _mm1(a, b, dims=None):
    if dims is None:
        dims = (((a.ndim - 1,), (0,)), ((), ()))
    return lax.dot_general(a.astype(jnp.bfloat16), b.astype(jnp.bfloat16),
                           dims, preferred_element_type=jnp.float32)


def _emb_gather(table, idx):
    info = plsc.get_sparse_core_info()
    nw = info.num_cores * info.num_subcores
    b_per_w = N // nw
    ch = 64
    mesh = plsc.VectorSubcoreMesh(core_axis_name="c", subcore_axis_name="s")

    @functools.partial(
        pl.kernel, mesh=mesh,
        out_type=jax.ShapeDtypeStruct((N, D), jnp.float32),
        scratch_types=[
            pltpu.VMEM((b_per_w,), jnp.int32),
            pltpu.VMEM((ch, D), jnp.float32),
            pltpu.SemaphoreType.DMA,
        ],
    )
    def k(table_hbm, idx_hbm, out_hbm, idx_v, rows_v, sem):
        wid = lax.axis_index("s") * info.num_cores + lax.axis_index("c")
        base = wid * b_per_w
        pltpu.sync_copy(idx_hbm.at[pl.ds(base, b_per_w)], idx_v)
        for c in range(b_per_w // ch):
            pltpu.async_copy(table_hbm.at[idx_v.at[pl.ds(c * ch, ch)]],
                             rows_v, sem).wait()
            pltpu.sync_copy(rows_v, out_hbm.at[pl.ds(base + c * ch, ch)])

    return k(table, idx)


def _qkv(x, wq, wk, wv):
    rb = 512

    def body(x_ref, wq_ref, wk_ref, wv_ref, q_ref, k_ref, v_ref):
        xv = x_ref[...].astype(jnp.bfloat16)
        q_ref[...] = _mm1(xv, wq_ref[...]).astype(jnp.bfloat16)
        k_ref[...] = _mm1(xv, wk_ref[...]).astype(jnp.bfloat16)
        v_ref[...] = _mm1(xv, wv_ref[...])

    w_spec = pl.BlockSpec((D, D), lambda i: (0, 0))
    r_spec = pl.BlockSpec((rb, D), lambda i: (i, 0))
    return pl.pallas_call(
        body,
        grid=(N // rb,),
        in_specs=[r_spec, w_spec, w_spec, w_spec],
        out_specs=[r_spec, r_spec, r_spec],
        out_shape=[jax.ShapeDtypeStruct((N, D), jnp.bfloat16),
                   jax.ShapeDtypeStruct((N, D), jnp.bfloat16),
                   jax.ShapeDtypeStruct((N, D), jnp.float32)],
    )(x, wq, wk, wv)


def _attention(q, k, v, kvalid):
    bq, bk = 1024, 1024
    nq, nk = N // bq, N // bk
    scale = 1.0 / math.sqrt(D)

    def body(q_ref, k_ref, v_ref, kv_ref, o_ref, m_s, l_s, acc_s):
        j = pl.program_id(1)

        @pl.when(j == 0)
        def _():
            m_s[...] = jnp.full_like(m_s, -jnp.inf)
            l_s[...] = jnp.zeros_like(l_s)
            acc_s[...] = jnp.zeros_like(acc_s)

        s = lax.dot_general(q_ref[...], k_ref[...],
                            (((1,), (1,)), ((), ())),
                            preferred_element_type=jnp.float32)
        s = jnp.where(kv_ref[...] > 0.0, s * scale, -jnp.inf)
        m_old = m_s[...]
        l_old = l_s[...]
        m_new = jnp.maximum(m_old, jnp.max(s, axis=1, keepdims=True))
        corr = jnp.exp(jnp.where(m_old == m_new, 0.0, m_old - m_new))
        p = jnp.exp(s - m_new)
        l_new = corr * l_old + jnp.sum(p, axis=1, keepdims=True)
        tot = _mm1(p, v_ref[...]) + (corr * l_old) * acc_s[...]
        acc_s[...] = tot * (1.0 / l_new)
        m_s[...] = m_new
        l_s[...] = l_new

        @pl.when(j == nk - 1)
        def _():
            o_ref[...] = acc_s[...]

    return pl.pallas_call(
        body,
        grid=(nq, nk),
        in_specs=[
            pl.BlockSpec((bq, D), lambda i, j: (i, 0)),
            pl.BlockSpec((bk, D), lambda i, j: (j, 0)),
            pl.BlockSpec((bk, D), lambda i, j: (j, 0)),
            pl.BlockSpec((1, bk), lambda i, j: (0, j)),
        ],
        out_specs=pl.BlockSpec((bq, D), lambda i, j: (i, 0)),
        out_shape=jax.ShapeDtypeStruct((N, D), jnp.float32),
        scratch_shapes=[
            pltpu.VMEM((bq, 1), jnp.float32),
            pltpu.VMEM((bq, 1), jnp.float32),
            pltpu.VMEM((bq, D), jnp.float32),
        ],
        compiler_params=pltpu.CompilerParams(
            dimension_semantics=("parallel", "arbitrary")),
    )(q, k, v, kvalid)


def _layernorm(h):
    m = jnp.mean(h, axis=-1, keepdims=True)
    d = h - m
    var = jnp.mean(d * d, axis=-1, keepdims=True)
    return d / jnp.sqrt(var + 1e-5)


def _post(attn, x, wo, w1, w2, wemit, validf):
    rb = 512

    def body(a_ref, x_ref, wo_ref, w1_ref, w2_ref, we_ref, vf_ref, em_ref):
        h = _layernorm(x_ref[...] + _mm1(a_ref[...], wo_ref[...]))
        g = _mm1(h, w1_ref[...]).astype(jnp.bfloat16)
        r = jnp.maximum(g, jnp.bfloat16(0))
        f = _layernorm(h + _mm1(r, w2_ref[...]))
        f = f * vf_ref[...]
        em_ref[...] = _mm1(f, we_ref[...])

    return pl.pallas_call(
        body,
        grid=(N // rb,),
        in_specs=[
            pl.BlockSpec((rb, D), lambda i: (i, 0)),
            pl.BlockSpec((rb, D), lambda i: (i, 0)),
            pl.BlockSpec((D, D), lambda i: (0, 0)),
            pl.BlockSpec((D, DFF), lambda i: (0, 0)),
            pl.BlockSpec((DFF, D), lambda i: (0, 0)),
            pl.BlockSpec((D, T), lambda i: (0, 0)),
            pl.BlockSpec((rb, 1), lambda i: (i, 0)),
        ],
        out_specs=pl.BlockSpec((rb, T), lambda i: (i, 0)),
        out_shape=jax.ShapeDtypeStruct((N, T), jnp.float32),
    )(attn, x, wo, w1, w2, wemit, validf)


def _crf(em_t, y_t, vf_t, trans, start_r, end_r):
    def body(em_ref, y_ref, vf_ref, tr_ref, st_ref, en_ref,
             loss_ref, path_ref, bp_ref):
        trans_v = tr_ref[...]
        etrans = jnp.exp(trans_v)
        start = st_ref[...]
        end = en_ref[...]
        iota_bt = lax.broadcasted_iota(jnp.int32, (B, T), 1)

        em0 = em_ref[pl.ds(0, 1)][0]
        alpha0 = start + em0
        delta0 = alpha0

        def fwd(t, carry):
            alpha, delta = carry
            em_step = em_ref[pl.ds(t, 1)][0]
            vt = vf_ref[pl.ds(t, 1)].reshape(B, 1) > 0.0
            m = jnp.max(alpha, axis=1, keepdims=True)
            p = jnp.exp(alpha - m)
            acc = _mm(p, etrans, precision=_HI)
            na = jnp.log(acc) + m + em_step
            alpha = jnp.where(vt, na, alpha)
            s3 = delta[:, :, None] + trans_v[None, :, :]
            mx = jnp.max(s3, axis=1)
            bp = jnp.argmax(s3, axis=1).astype(jnp.int32)
            bp_ref[pl.ds(t, 1)] = jnp.where(vt, bp, iota_bt)[None]
            delta = jnp.where(vt, mx + em_step, delta)
            return alpha, delta

        alpha, delta = lax.fori_loop(1, S, fwd, (alpha0, delta0))

        em_all = em_ref[...]
        y = y_ref[...]
        vf = vf_ref[...]
        iota3 = lax.broadcasted_iota(jnp.int32, (S, B, T), 2)
        yoh = (iota3 == y[:, :, None]).astype(jnp.float32)
        em_y = jnp.sum(em_all * yoh, axis=2)
        g_em = jnp.sum(em_y * vf, axis=0)
        st_term = jnp.sum(start * (iota_bt == y[0][:, None]), axis=1)
        rows = _mm(yoh[:-1].reshape((S - 1) * B, T), trans_v, precision=_HI)
        sel = jnp.sum(rows * yoh[1:].reshape((S - 1) * B, T),
                      axis=1).reshape(S - 1, B)
        g_tr = jnp.sum(sel * vf[1:], axis=0)
        lengths = jnp.sum(vf, axis=0).astype(jnp.int32)
        iota_sb = lax.broadcasted_iota(jnp.int32, (S, B), 0)
        ohlen = (iota_sb == (lengths - 1)[None, :]).astype(jnp.int32)
        ylast = jnp.sum(y * ohlen, axis=0)
        en_term = jnp.sum(end * (iota_bt == ylast[:, None]), axis=1)
        gold = g_em + st_term + g_tr + en_term
        ae = alpha + end
        mz = jnp.max(ae, axis=1, keepdims=True)
        logz = mz[:, 0] + jnp.log(jnp.sum(jnp.exp(ae - mz), axis=1))
        loss_ref[...] = jnp.mean(logz - gold).reshape(1, 1)

        de = delta + end
        mx0 = jnp.max(de, axis=1, keepdims=True)
        last0 = jnp.min(jnp.where(de >= mx0, iota_bt, T),
                        axis=1, keepdims=True)
        path_ref[pl.ds(S - 1, 1)] = last0.reshape(1, B)

        def back(i, last):
            t = S - 2 - i
            bpt = bp_ref[pl.ds(t + 1, 1)][0]
            eq = (iota_bt == last).astype(jnp.int32)
            nxt = jnp.sum(bpt * eq, axis=1, keepdims=True)
            path_ref[pl.ds(t, 1)] = nxt.reshape(1, B)
            return nxt

        lax.fori_loop(0, S - 1, back, last0)

    loss, path_t = pl.pallas_call(
        body,
        in_specs=[pl.BlockSpec(memory_space=pltpu.VMEM)] * 6,
        out_specs=[pl.BlockSpec(memory_space=pltpu.VMEM)] * 2,
        out_shape=[jax.ShapeDtypeStruct((1, 1), jnp.float32),
                   jax.ShapeDtypeStruct((S, B), jnp.int32)],
        scratch_shapes=[pltpu.VMEM((S, B, T), jnp.int32)],
    )(em_t, y_t, vf_t, trans, start_r, end_r)
    return loss, path_t


def kernel(src, y, mask, use_gpu, embedding_matrix, Wq, Wk, Wv, Wo,
           W1, W2, W_emit, trans, start, end):
    valid = jnp.logical_not(mask)
    valid_flat = valid.reshape(-1)
    validf = valid_flat.astype(jnp.float32)
    src_flat = src.reshape(-1).astype(jnp.int32)

    x = _emb_gather(embedding_matrix, src_flat)
    q, k, v = _qkv(x, Wq, Wk, Wv)
    attn = _attention(q, k, v, validf.reshape(1, N))
    em = _post(attn, x, Wo, W1, W2, W_emit, validf.reshape(N, 1))

    em_t = em.reshape(B, S, T).transpose(1, 0, 2)
    y_t = y.astype(jnp.int32).transpose(1, 0)
    vf_t = validf.reshape(B, S).transpose(1, 0)
    loss, path_t = _crf(em_t, y_t, vf_t, trans,
                        start.reshape(1, T), end.reshape(1, T))
    return loss[0, 0], path_t.transpose(1, 0)


# speedup vs baseline: 1.1180x; 1.0148x over previous
import functools
import math

import jax
import jax.numpy as jnp
from jax import lax
from jax.experimental import pallas as pl
from jax.experimental.pallas import tpu as pltpu
from jax.experimental.pallas import tpu_sc as plsc

B, S, D, V, T, DFF = 8, 512, 1024, 32000, 64, 2048
N = B * S

_HI = jax.lax.Precision.HIGHEST
_NEG = -1e30


def _mm(a, b, precision=jax.lax.Precision.DEFAULT):
    return lax.dot_general(a, b, (((a.ndim - 1,), (0,)), ((), ())),
                           precision=precision,
                           preferred_element_type=jnp.float32)


def _mm1(a, b, dims=None):
    if dims is None:
        dims = (((a.ndim - 1,), (0,)), ((), ()))
    return lax.dot_general(a.astype(jnp.bfloat16), b.astype(jnp.bfloat16),
                           dims, preferred_element_type=jnp.float32)


def _emb_gather(table, idx):
    info = plsc.get_sparse_core_info()
    nw = info.num_cores * info.num_subcores
    b_per_w = N // nw
    ch = 64
    mesh = plsc.VectorSubcoreMesh(core_axis_name="c", subcore_axis_name="s")

    @functools.partial(
        pl.kernel, mesh=mesh,
        out_type=jax.ShapeDtypeStruct((N, D), jnp.float32),
        scratch_types=[
            pltpu.VMEM((b_per_w,), jnp.int32),
            pltpu.VMEM((ch, D), jnp.float32),
            pltpu.SemaphoreType.DMA,
        ],
    )
    def k(table_hbm, idx_hbm, out_hbm, idx_v, rows_v, sem):
        wid = lax.axis_index("s") * info.num_cores + lax.axis_index("c")
        base = wid * b_per_w
        pltpu.sync_copy(idx_hbm.at[pl.ds(base, b_per_w)], idx_v)
        for c in range(b_per_w // ch):
            pltpu.async_copy(table_hbm.at[idx_v.at[pl.ds(c * ch, ch)]],
                             rows_v, sem).wait()
            pltpu.sync_copy(rows_v, out_hbm.at[pl.ds(base + c * ch, ch)])

    return k(table, idx)


def _qkv(x, wq, wk, wv):
    rb = 512

    def body(x_ref, wq_ref, wk_ref, wv_ref, q_ref, k_ref, v_ref):
        xv = x_ref[...].astype(jnp.bfloat16)
        q_ref[...] = _mm1(xv, wq_ref[...]).astype(jnp.bfloat16)
        k_ref[...] = _mm1(xv, wk_ref[...]).astype(jnp.bfloat16)
        v_ref[...] = _mm1(xv, wv_ref[...])

    w_spec = pl.BlockSpec((D, D), lambda i: (0, 0))
    r_spec = pl.BlockSpec((rb, D), lambda i: (i, 0))
    return pl.pallas_call(
        body,
        grid=(N // rb,),
        in_specs=[r_spec, w_spec, w_spec, w_spec],
        out_specs=[r_spec, r_spec, r_spec],
        out_shape=[jax.ShapeDtypeStruct((N, D), jnp.bfloat16),
                   jax.ShapeDtypeStruct((N, D), jnp.bfloat16),
                   jax.ShapeDtypeStruct((N, D), jnp.float32)],
    )(x, wq, wk, wv)


def _attention(q, k, v, kvalid):
    bq, bk = 1024, 1024
    nq, nk = N // bq, N // bk
    scale = 1.0 / math.sqrt(D)

    def body(q_ref, k_ref, v_ref, kv_ref, o_ref, m_s, l_s, acc_s):
        j = pl.program_id(1)

        @pl.when(j == 0)
        def _():
            m_s[...] = jnp.full_like(m_s, -jnp.inf)
            l_s[...] = jnp.zeros_like(l_s)
            acc_s[...] = jnp.zeros_like(acc_s)

        s = lax.dot_general(q_ref[...], k_ref[...],
                            (((1,), (1,)), ((), ())),
                            preferred_element_type=jnp.float32)
        s = jnp.where(kv_ref[...] > 0.0, s * scale, -jnp.inf)
        m_old = m_s[...]
        l_old = l_s[...]
        m_new = jnp.maximum(m_old, jnp.max(s, axis=1, keepdims=True))
        corr = jnp.exp(jnp.where(m_old == m_new, 0.0, m_old - m_new))
        p = jnp.exp(s - m_new)
        l_new = corr * l_old + jnp.sum(p, axis=1, keepdims=True)
        tot = _mm1(p, v_ref[...]) + (corr * l_old) * acc_s[...]
        acc_s[...] = tot * (1.0 / l_new)
        m_s[...] = m_new
        l_s[...] = l_new

        @pl.when(j == nk - 1)
        def _():
            o_ref[...] = acc_s[...]

    return pl.pallas_call(
        body,
        grid=(nq, nk),
        in_specs=[
            pl.BlockSpec((bq, D), lambda i, j: (i, 0)),
            pl.BlockSpec((bk, D), lambda i, j: (j, 0)),
            pl.BlockSpec((bk, D), lambda i, j: (j, 0)),
            pl.BlockSpec((1, bk), lambda i, j: (0, j)),
        ],
        out_specs=pl.BlockSpec((bq, D), lambda i, j: (i, 0)),
        out_shape=jax.ShapeDtypeStruct((N, D), jnp.float32),
        scratch_shapes=[
            pltpu.VMEM((bq, 1), jnp.float32),
            pltpu.VMEM((bq, 1), jnp.float32),
            pltpu.VMEM((bq, D), jnp.float32),
        ],
        compiler_params=pltpu.CompilerParams(
            dimension_semantics=("parallel", "arbitrary")),
    )(q, k, v, kvalid)


def _layernorm(h):
    m = jnp.mean(h, axis=-1, keepdims=True)
    d = h - m
    var = jnp.mean(d * d, axis=-1, keepdims=True)
    return d / jnp.sqrt(var + 1e-5)


def _post(attn, x, wo, w1, w2, wemit, validf):
    rb = 512

    def body(a_ref, x_ref, wo_ref, w1_ref, w2_ref, we_ref, vf_ref, em_ref):
        h = _layernorm(x_ref[...] + _mm1(a_ref[...], wo_ref[...]))
        g = _mm1(h, w1_ref[...]).astype(jnp.bfloat16)
        r = jnp.maximum(g, jnp.bfloat16(0))
        f = _layernorm(h + _mm1(r, w2_ref[...]))
        f = f * vf_ref[...]
        em_ref[...] = _mm1(f, we_ref[...])

    return pl.pallas_call(
        body,
        grid=(N // rb,),
        in_specs=[
            pl.BlockSpec((rb, D), lambda i: (i, 0)),
            pl.BlockSpec((rb, D), lambda i: (i, 0)),
            pl.BlockSpec((D, D), lambda i: (0, 0)),
            pl.BlockSpec((D, DFF), lambda i: (0, 0)),
            pl.BlockSpec((DFF, D), lambda i: (0, 0)),
            pl.BlockSpec((D, T), lambda i: (0, 0)),
            pl.BlockSpec((rb, 1), lambda i: (i, 0)),
        ],
        out_specs=pl.BlockSpec((rb, T), lambda i: (i, 0)),
        out_shape=jax.ShapeDtypeStruct((N, T), jnp.float32),
    )(attn, x, wo, w1, w2, wemit, validf)


def _crf(em_t, y_t, vf_t, trans, start_r, end_r):
    def body(em_ref, y_ref, vf_ref, tr_ref, st_ref, en_ref,
             loss_ref, path_ref, bp_ref):
        trans_v = tr_ref[...]
        etrans = jnp.exp(trans_v)
        start = st_ref[...]
        end = en_ref[...]
        iota_bt = lax.broadcasted_iota(jnp.int32, (B, T), 1)

        em0 = em_ref[pl.ds(0, 1)][0]
        alpha0 = start + em0
        delta0 = alpha0

        def fwd(t, carry):
            alpha, delta = carry
            em_step = em_ref[pl.ds(t, 1)][0]
            vt = vf_ref[pl.ds(t, 1)].reshape(B, 1) > 0.0
            m = jnp.max(alpha, axis=1, keepdims=True)
            p = jnp.exp(alpha - m)
            acc = _mm(p, etrans, precision=_HI)
            na = jnp.log(acc) + m + em_step
            alpha = jnp.where(vt, na, alpha)
            s3 = delta[:, :, None] + trans_v[None, :, :]
            mx = jnp.max(s3, axis=1)
            bp = jnp.argmax(s3, axis=1).astype(jnp.int32)
            bp_ref[pl.ds(t, 1)] = jnp.where(vt, bp, iota_bt)[None]
            delta = jnp.where(vt, mx + em_step, delta)
            return alpha, delta

        lengths_i = jnp.sum(vf_ref[...], axis=0).astype(jnp.int32)
        lmax = jnp.max(lengths_i)
        alpha, delta = lax.fori_loop(1, lmax, fwd, (alpha0, delta0))

        em_all = em_ref[...]
        y = y_ref[...]
        vf = vf_ref[...]
        iota3 = lax.broadcasted_iota(jnp.int32, (S, B, T), 2)
        yoh = (iota3 == y[:, :, None]).astype(jnp.float32)
        em_y = jnp.sum(em_all * yoh, axis=2)
        g_em = jnp.sum(em_y * vf, axis=0)
        st_term = jnp.sum(start * (iota_bt == y[0][:, None]), axis=1)
        rows = _mm(yoh[:-1].reshape((S - 1) * B, T), trans_v, precision=_HI)
        sel = jnp.sum(rows * yoh[1:].reshape((S - 1) * B, T),
                      axis=1).reshape(S - 1, B)
        g_tr = jnp.sum(sel * vf[1:], axis=0)
        iota_sb = lax.broadcasted_iota(jnp.int32, (S, B), 0)
        ohlen = (iota_sb == (lengths_i - 1)[None, :]).astype(jnp.int32)
        ylast = jnp.sum(y * ohlen, axis=0)
        en_term = jnp.sum(end * (iota_bt == ylast[:, None]), axis=1)
        gold = g_em + st_term + g_tr + en_term
        ae = alpha + end
        mz = jnp.max(ae, axis=1, keepdims=True)
        logz = mz[:, 0] + jnp.log(jnp.sum(jnp.exp(ae - mz), axis=1))
        loss_ref[...] = jnp.mean(logz - gold).reshape(1, 1)

        de = delta + end
        mx0 = jnp.max(de, axis=1, keepdims=True)
        last0 = jnp.min(jnp.where(de >= mx0, iota_bt, T),
                        axis=1, keepdims=True)
        path_ref[...] = jnp.broadcast_to(last0.reshape(1, B), (S, B))

        def back(i, last):
            t = lmax - 2 - i
            bpt = bp_ref[pl.ds(t + 1, 1)][0]
            eq = (iota_bt == last).astype(jnp.int32)
            nxt = jnp.sum(bpt * eq, axis=1, keepdims=True)
            path_ref[pl.ds(t, 1)] = nxt.reshape(1, B)
            return nxt

        lax.fori_loop(0, lmax - 1, back, last0)

    loss, path_t = pl.pallas_call(
        body,
        in_specs=[pl.BlockSpec(memory_space=pltpu.VMEM)] * 6,
        out_specs=[pl.BlockSpec(memory_space=pltpu.VMEM)] * 2,
        out_shape=[jax.ShapeDtypeStruct((1, 1), jnp.float32),
                   jax.ShapeDtypeStruct((S, B), jnp.int32)],
        scratch_shapes=[pltpu.VMEM((S, B, T), jnp.int32)],
    )(em_t, y_t, vf_t, trans, start_r, end_r)
    return loss, path_t


def kernel(src, y, mask, use_gpu, embedding_matrix, Wq, Wk, Wv, Wo,
           W1, W2, W_emit, trans, start, end):
    valid = jnp.logical_not(mask)
    valid_flat = valid.reshape(-1)
    validf = valid_flat.astype(jnp.float32)
    src_flat = src.reshape(-1).astype(jnp.int32)

    x = _emb_gather(embedding_matrix, src_flat)
    q, k, v = _qkv(x, Wq, Wk, Wv)
    attn = _attention(q, k, v, validf.reshape(1, N))
    em = _post(attn, x, Wo, W1, W2, W_emit, validf.reshape(N, 1))

    em_t = em.reshape(B, S, T).transpose(1, 0, 2)
    y_t = y.astype(jnp.int32).transpose(1, 0)
    vf_t = validf.reshape(B, S).transpose(1, 0)
    loss, path_t = _crf(em_t, y_t, vf_t, trans,
                        start.reshape(1, T), end.reshape(1, T))
    return loss[0, 0], path_t.transpose(1, 0)
